# baseline (speedup 1.0000x reference)
import functools

import jax
import jax.numpy as jnp
from jax.experimental import pallas as pl

N = 10000
F = 128
BD = 128
BM = 400


def _prep_kernel(x_ref, w_ref, adj_diag_ref, b_ref, support_ref, vb_ref):
    i = pl.program_id(0)
    rows = jax.lax.broadcasted_iota(jnp.int32, (BD, BD), 0)
    cols = jax.lax.broadcasted_iota(jnp.int32, (BD, BD), 1)
    valid = (i * BD + rows) < N
    s = jnp.dot(x_ref[...], w_ref[...], preferred_element_type=jnp.float32)
    s = jnp.where(valid, s, 0.0)
    support_ref[...] = s
    dmat = jnp.where((rows == cols) & valid, adj_diag_ref[...], 0.0)
    d_row = jnp.sum(dmat, axis=0, keepdims=True)
    partial = jnp.dot(d_row, s, preferred_element_type=jnp.float32)

    @pl.when(i == 0)
    def _init():
        vb_ref[...] = partial + b_ref[...]

    @pl.when(i != 0)
    def _acc():
        vb_ref[...] += partial


def _main_kernel(adj_ref, support_ref, vb_ref, out_ref):
    acc = jnp.dot(adj_ref[...], support_ref[...],
                  preferred_element_type=jnp.float32)
    out_ref[...] = acc + vb_ref[...]


@jax.jit
def kernel(input, adj, W, b):
    x = input
    b2 = b.reshape(1, F)

    support, vb = pl.pallas_call(
        _prep_kernel,
        grid=(pl.cdiv(N, BD),),
        in_specs=[
            pl.BlockSpec((BD, F), lambda i: (i, 0)),
            pl.BlockSpec((F, F), lambda i: (0, 0)),
            pl.BlockSpec((BD, BD), lambda i: (i, i)),
            pl.BlockSpec((1, F), lambda i: (0, 0)),
        ],
        out_specs=[
            pl.BlockSpec((BD, F), lambda i: (i, 0)),
            pl.BlockSpec((1, F), lambda i: (0, 0)),
        ],
        out_shape=[
            jax.ShapeDtypeStruct((N, F), jnp.float32),
            jax.ShapeDtypeStruct((1, F), jnp.float32),
        ],
    )(x, W, adj, b2)

    out = pl.pallas_call(
        _main_kernel,
        grid=(N // BM,),
        in_specs=[
            pl.BlockSpec((BM, N), lambda i: (i, 0)),
            pl.BlockSpec((N, F), lambda i: (0, 0)),
            pl.BlockSpec((1, F), lambda i: (0, 0)),
        ],
        out_specs=pl.BlockSpec((BM, F), lambda i: (i, 0)),
        out_shape=jax.ShapeDtypeStruct((N, F), jnp.float32),
    )(adj, support, vb)

    return out


# speedup vs baseline: 1.1642x; 1.1642x over previous
import jax
import jax.numpy as jnp
from jax.experimental import pallas as pl

N = 10000
F = 128
BM = 400
NB = N // BM
WIN = 512


def _prep_kernel(x_ref, w_ref, sup16_ref):
    s = jnp.dot(x_ref[...], w_ref[...], preferred_element_type=jnp.float32)
    sup16_ref[...] = s.astype(jnp.bfloat16)


def _main_kernel(adj_ref, sup16_ref, sup16_blk_ref, y_ref, rsum_ref):
    i = pl.program_id(0)
    a = adj_ref[...]
    a16 = a.astype(jnp.bfloat16)
    y_ref[...] = jnp.dot(a16, sup16_ref[...], preferred_element_type=jnp.float32)

    k = jax.lax.broadcasted_iota(jnp.int32, (BM, N), 0)
    j = jax.lax.broadcasted_iota(jnp.int32, (BM, N), 1)
    d_col = jnp.sum(jnp.where(j == k + i * BM, a, 0.0), axis=1, keepdims=True)
    s_blk = sup16_blk_ref[...].astype(jnp.float32)
    contrib = jnp.sum(d_col * s_blk, axis=0, keepdims=True)

    @pl.when(i == 0)
    def _():
        rsum_ref[...] = jnp.zeros_like(rsum_ref)

    rsum_ref[...] += contrib


def _post_kernel(y_ref, rsum_ref, b_ref, out_ref):
    out_ref[...] = y_ref[...] + rsum_ref[...] + b_ref[...]


@jax.jit
def kernel(input, adj, W, b):
    b2 = b.reshape(1, F)
    sup16 = pl.pallas_call(
        _prep_kernel,
        in_specs=[
            pl.BlockSpec((N, F), lambda: (0, 0)),
            pl.BlockSpec((F, F), lambda: (0, 0)),
        ],
        out_specs=pl.BlockSpec((N, F), lambda: (0, 0)),
        out_shape=jax.ShapeDtypeStruct((N, F), jnp.bfloat16),
    )(input, W)

    y, rsum = pl.pallas_call(
        _main_kernel,
        grid=(NB,),
        in_specs=[
            pl.BlockSpec((BM, N), lambda i: (i, 0)),
            pl.BlockSpec((N, F), lambda i: (0, 0)),
            pl.BlockSpec((BM, F), lambda i: (i, 0)),
        ],
        out_specs=[
            pl.BlockSpec((BM, F), lambda i: (i, 0)),
            pl.BlockSpec((1, F), lambda i: (0, 0)),
        ],
        out_shape=[
            jax.ShapeDtypeStruct((N, F), jnp.float32),
            jax.ShapeDtypeStruct((1, F), jnp.float32),
        ],
    )(adj, sup16, sup16)

    out = pl.pallas_call(
        _post_kernel,
        grid=(NB,),
        in_specs=[
            pl.BlockSpec((BM, F), lambda i: (i, 0)),
            pl.BlockSpec((1, F), lambda i: (0, 0)),
            pl.BlockSpec((1, F), lambda i: (0, 0)),
        ],
        out_specs=pl.BlockSpec((BM, F), lambda i: (i, 0)),
        out_shape=jax.ShapeDtypeStruct((N, F), jnp.float32),
    )(y, rsum, b2)
    return out
